# baseline (speedup 1.0000x reference)
import functools

import jax
import jax.numpy as jnp
from jax import lax
from jax.experimental import pallas as pl
from jax.experimental.pallas import tpu as pltpu
from jax.experimental.pallas import tpu_sc as plsc

N_EMB = 64
BATCH = 16384
NC, NS, L = 2, 16, 16
NW = NC * NS
BPW = BATCH // NW
CHUNK = 128
NCH = BPW // CHUNK
NVR = N_EMB // L


def _gmf_body(users_hbm, items_hbm, utab_hbm, itab_hbm, par_hbm, out_hbm,
              uidx_v, iidx_v, urows_v, irows_v, par_v, out_v, *sems):
    wid = lax.axis_index("s") * NC + lax.axis_index("c")
    base = wid * BPW

    pltpu.sync_copy(users_hbm.at[wid], uidx_v)
    pltpu.sync_copy(items_hbm.at[wid], iidx_v)
    pltpu.sync_copy(par_hbm, par_v)

    copies = []
    for c in range(NCH):
        cu = pltpu.async_copy(
            utab_hbm.at[uidx_v.at[c]],
            urows_v.at[pl.ds(c * CHUNK, CHUNK)],
            sems[2 * c])
        ci = pltpu.async_copy(
            itab_hbm.at[iidx_v.at[c]],
            irows_v.at[pl.ds(c * CHUNK, CHUNK)],
            sems[2 * c + 1])
        copies.append((cu, ci))

    w = [par_v[pl.ds(k * L, L)] for k in range(NVR)]
    bias = par_v[pl.ds(N_EMB, L)]
    lane = lax.iota(jnp.int32, L)

    groups_per_chunk = CHUNK // L

    for c in range(NCH):
        copies[c][0].wait()
        copies[c][1].wait()

        def body(g, _, c=c):
            row0 = c * CHUNK + g * L
            acc = jnp.zeros((L,), jnp.float32)
            for j in range(L):
                r = row0 + j
                t = jnp.zeros((L,), jnp.float32)
                for k in range(NVR):
                    u = urows_v[r, pl.ds(k * L, L)]
                    v = irows_v[r, pl.ds(k * L, L)]
                    t = t + (u * v) * w[k]
                s = jnp.sum(t)
                acc = jnp.where(lane == j, s, acc)
            z = acc + bias
            p = 1.0 / (1.0 + jnp.exp(-z))
            out_v[pl.ds(row0, L)] = p
            return _

        lax.fori_loop(0, groups_per_chunk, body, 0)

    pltpu.sync_copy(out_v, out_hbm.at[pl.ds(base, BPW)])


@jax.jit
def _gmf(users3, items3, user_table, item_table, par):
    mesh = plsc.VectorSubcoreMesh(core_axis_name="c", subcore_axis_name="s",
                                  num_cores=NC, num_subcores=NS)
    scratch = [
        pltpu.VMEM((NCH, CHUNK), jnp.int32),
        pltpu.VMEM((NCH, CHUNK), jnp.int32),
        pltpu.VMEM((BPW, N_EMB), jnp.float32),
        pltpu.VMEM((BPW, N_EMB), jnp.float32),
        pltpu.VMEM((5 * L,), jnp.float32),
        pltpu.VMEM((BPW,), jnp.float32),
    ] + [pltpu.SemaphoreType.DMA] * (2 * NCH)
    run = pl.kernel(
        _gmf_body,
        out_type=jax.ShapeDtypeStruct((BATCH,), jnp.float32),
        mesh=mesh,
        scratch_types=scratch,
        compiler_params=pltpu.CompilerParams(needs_layout_passes=False,
                                             use_tc_tiling_on_sc=False),
    )
    return run(users3, items3, user_table, item_table, par)


def kernel(users, items, user_table, item_table, W, b):
    users3 = users.reshape(NW, NCH, CHUNK)
    items3 = items.reshape(NW, NCH, CHUNK)
    par = jnp.concatenate(
        [W.reshape(-1), jnp.full((L,), b[0], jnp.float32)])
    out = _gmf(users3, items3, user_table, item_table, par)
    return out.reshape(BATCH, 1)
